# baseline (speedup 1.0000x reference)
import functools

import jax
import jax.numpy as jnp
from jax import lax
from jax.experimental import pallas as pl
from jax.experimental.pallas import tpu as pltpu
from jax.experimental.pallas import tpu_sc as plsc

N = 10000
E = 320000
D = 128

NC = 2
NS = 16
NW = NC * NS

C = 128
CHUNKS = 80
EPT = C * CHUNKS
E_PAD = EPT * NW
TRASH = N
N_ACC = 10240
RPT = N_ACC // NS
BPT = RPT // C
NBUF = 2

_mesh = plsc.VectorSubcoreMesh(
    core_axis_name="c", subcore_axis_name="s", num_cores=NC, num_subcores=NS)


def _ids():
    cid = lax.axis_index("c")
    sid = lax.axis_index("s")
    return cid, sid, sid * NC + cid


def _zero_acc(zeros_hbm, stage_v, acc, sid):
    pltpu.sync_copy(zeros_hbm.at[pl.ds(0, C)], stage_v)
    for b in range(BPT):
        pltpu.sync_copy(stage_v, acc.at[pl.ds(sid * RPT + b * C, C)])


def _writeback(acc, stage_v, out_hbm, cid, sid):
    for b in range(BPT):
        r0 = sid * RPT + b * C
        pltpu.sync_copy(acc.at[pl.ds(r0, C)], stage_v)
        pltpu.sync_copy(stage_v, out_hbm.at[pl.ds(cid * N_ACC + r0, C)])


def _preload_cols(colp_hbm, col_b, wid):
    pltpu.sync_copy(colp_hbm.at[pl.ds(wid * CHUNKS, CHUNKS)], col_b)


def _mask_cols(col_b, rbuf, i):
    for j in range(C // 16):
        sl = pl.ds(j * 16, 16)
        r = rbuf[sl]
        c = col_b[i, sl]
        col_b[i, sl] = jnp.where(r == c, TRASH, c)


K0 = 120
K1 = 40
KMAX = max(K0, K1)


def _agg_pipeline(h_hbm, rowp_hbm, acc, col_b, rbufs, rows,
                  isems, gsems, ssems, chunk0, K):
    rounds = K // NBUF

    for b in range(NBUF):
        pltpu.sync_copy(rowp_hbm.at[chunk0 + b], rbufs[b])
        _mask_cols(col_b, rbufs[b], b)
        pltpu.async_copy(h_hbm.at[rbufs[b]], rows[b], gsems[b])

    def steady(o, _):
        for b in range(NBUF):
            ic = o * NBUF + b
            pltpu.make_async_copy(h_hbm.at[rbufs[b]], rows[b],
                                  gsems[b]).wait()
            pltpu.async_copy(rows[b], acc.at[col_b.at[ic]], ssems[b],
                             add=True)
            pltpu.async_copy(rowp_hbm.at[chunk0 + ic + NBUF],
                             rbufs[b], isems[b])
        for b in range(NBUF):
            ic = o * NBUF + b
            pltpu.make_async_copy(rowp_hbm.at[0], rbufs[b],
                                  isems[b]).wait()
            _mask_cols(col_b, rbufs[b], ic + NBUF)
            pltpu.make_async_copy(rows[b], acc.at[col_b.at[ic]],
                                  ssems[b]).wait()
            pltpu.async_copy(h_hbm.at[rbufs[b]], rows[b], gsems[b])
        return ()

    lax.fori_loop(0, rounds - 1, steady, (), unroll=False)

    for b in range(NBUF):
        ic = (rounds - 1) * NBUF + b
        pltpu.make_async_copy(h_hbm.at[rbufs[b]], rows[b], gsems[b]).wait()
        pltpu.async_copy(rows[b], acc.at[col_b.at[ic]], ssems[b], add=True)
    for b in range(NBUF):
        ic = (rounds - 1) * NBUF + b
        pltpu.make_async_copy(rows[b], acc.at[col_b.at[ic]], ssems[b]).wait()


def _sc_agg_body(h_hbm, rowp_hbm, colp_hbm, zeros_hbm, parts_out,
                 acc, col_b, rbufs, rows, isems, gsems, ssems):
    cid, sid, wid = _ids()

    @pl.when(cid == 0)
    def _():
        _zero_acc(zeros_hbm, rows[0], acc, sid)
        for half in range(2):
            chunk0 = sid * (2 * CHUNKS) + half * CHUNKS
            pltpu.sync_copy(colp_hbm.at[pl.ds(chunk0, CHUNKS)], col_b)
            _agg_pipeline(h_hbm, rowp_hbm, acc, col_b, rbufs, rows,
                          isems, gsems, ssems, chunk0, CHUNKS)
        _writeback_single(acc, rows[0], parts_out, sid)


def _writeback_single(acc, stage_v, out_hbm, sid):
    for b in range(BPT):
        r0 = sid * RPT + b * C
        pltpu.sync_copy(acc.at[pl.ds(r0, C)], stage_v)
        pltpu.sync_copy(stage_v, out_hbm.at[pl.ds(r0, C)])


def _sc_deg_body(rowp_hbm, colp_hbm, zeros_hbm, ones_hbm, deg_out,
                 acc, row_b, col_b, ones_v, ssems):
    cid, sid, wid = _ids()
    _zero_acc(zeros_hbm, ones_v, acc, sid)
    pltpu.sync_copy(ones_hbm, ones_v)
    pltpu.sync_copy(rowp_hbm.at[pl.ds(wid * CHUNKS, CHUNKS)], row_b)
    _preload_cols(colp_hbm, col_b, wid)

    def mask_all(i, _):
        _mask_cols(col_b, row_b.at[i], i)
        return ()

    lax.fori_loop(0, CHUNKS, mask_all, (), unroll=False)
    plsc.subcore_barrier()

    def steady(o, _):
        for b in range(NBUF):
            i = o * NBUF + b
            pltpu.async_copy(ones_v, acc.at[col_b.at[i]], ssems[b], add=True)
        for b in range(NBUF):
            i = o * NBUF + b
            pltpu.make_async_copy(ones_v, acc.at[col_b.at[i]],
                                  ssems[b]).wait()
        return ()

    lax.fori_loop(0, CHUNKS // NBUF, steady, (), unroll=False)
    plsc.subcore_barrier()
    _writeback(acc, ones_v, deg_out, cid, sid)


_sc_agg = pl.kernel(
    _sc_agg_body,
    out_type=[jax.ShapeDtypeStruct((N_ACC, D), jnp.float32)],
    mesh=_mesh,
    scratch_types=[
        pltpu.VMEM_SHARED((N_ACC, D), jnp.float32),
        pltpu.VMEM((CHUNKS, C), jnp.int32),
        [pltpu.VMEM((C,), jnp.int32) for _ in range(NBUF)],
        [pltpu.VMEM((C, D), jnp.float32) for _ in range(NBUF)],
        [pltpu.SemaphoreType.DMA for _ in range(NBUF)],
        [pltpu.SemaphoreType.DMA for _ in range(NBUF)],
        [pltpu.SemaphoreType.DMA for _ in range(NBUF)],
    ],
)

_sc_deg = pl.kernel(
    _sc_deg_body,
    out_type=[jax.ShapeDtypeStruct((NC * N_ACC, D), jnp.float32)],
    mesh=_mesh,
    scratch_types=[
        pltpu.VMEM_SHARED((N_ACC, D), jnp.float32),
        pltpu.VMEM((CHUNKS, C), jnp.int32),
        pltpu.VMEM((CHUNKS, C), jnp.int32),
        pltpu.VMEM((C, D), jnp.float32),
        [pltpu.SemaphoreType.DMA for _ in range(NBUF)],
    ],
)


def _tc_body(act, parts_ref, degp_ref, h_ref, wo_ref, b_ref, wr_ref, o_ref):
    deg = degp_ref[0] + degp_ref[1] + 1.0
    dinv = 1.0 / deg
    h = h_ref[...]
    agg = (parts_ref[...] + h) * dinv
    y = (lax.dot_general(agg, wo_ref[...], (((1,), (0,)), ((), ())),
                         preferred_element_type=jnp.float32)
         + b_ref[...]
         + lax.dot_general(h, wr_ref[...], (((1,), (0,)), ((), ())),
                           preferred_element_type=jnp.float32))
    y = jnp.maximum(y, 0.0)
    if act == "relu":
        o_ref[...] = y
    else:
        m = jnp.max(y, axis=-1, keepdims=True)
        z = y - m
        o_ref[...] = z - jnp.log(jnp.sum(jnp.exp(z), axis=-1, keepdims=True))


RB = 1024


def _tc_layer(parts, degp, h, W_out, b_out, W_root, act):
    return pl.pallas_call(
        functools.partial(_tc_body, act),
        grid=(N_ACC // RB,),
        in_specs=[
            pl.BlockSpec((RB, D), lambda i: (i, 0)),
            pl.BlockSpec((NC, RB, D), lambda i: (0, i, 0)),
            pl.BlockSpec((RB, D), lambda i: (i, 0)),
            pl.BlockSpec((D, D), lambda i: (0, 0)),
            pl.BlockSpec((1, D), lambda i: (0, 0)),
            pl.BlockSpec((D, D), lambda i: (0, 0)),
        ],
        out_specs=pl.BlockSpec((RB, D), lambda i: (i, 0)),
        out_shape=jax.ShapeDtypeStruct((N_ACC, D), jnp.float32),
    )(parts, degp, h, W_out, b_out.reshape(1, D), W_root)


def kernel(x, edge_index, W_out0, b_out0, W_root0, W_out1, b_out1, W_root1,
           W_out2, b_out2, W_root2):
    row = edge_index[0]
    col = edge_index[1]
    pad = E_PAD - E
    rowp = jnp.concatenate([row, jnp.zeros((pad,), jnp.int32)])
    colp = jnp.concatenate([col, jnp.full((pad,), TRASH, jnp.int32)])
    rowp = rowp.reshape(NW * CHUNKS, C)
    colp = colp.reshape(NW * CHUNKS, C)
    zeros = jnp.zeros((N_ACC, D), jnp.float32)
    ones = jnp.ones((C, D), jnp.float32)
    xp = jnp.concatenate([x, jnp.zeros((N_ACC - N, D), jnp.float32)])

    (degp,) = _sc_deg(rowp, colp, zeros, ones)
    degp = degp.reshape(NC, N_ACC, D)
    (parts0,) = _sc_agg(xp, rowp, colp, zeros)
    h1 = _tc_layer(parts0, degp, xp, W_out0, b_out0, W_root0, "relu")
    (parts1,) = _sc_agg(h1, rowp, colp, zeros)
    h2 = _tc_layer(parts1, degp, h1, W_out1, b_out1, W_root1, "relu")
    (parts2,) = _sc_agg(h2, rowp, colp, zeros)
    out = _tc_layer(parts2, degp, h2, W_out2, b_out2, W_root2, "logsoftmax")
    return out[:N]


# speedup vs baseline: 1.3211x; 1.3211x over previous
import functools

import jax
import jax.numpy as jnp
from jax import lax
from jax.experimental import pallas as pl
from jax.experimental.pallas import tpu as pltpu
from jax.experimental.pallas import tpu_sc as plsc

N = 10000
E = 320000
D = 128

NC = 2
NS = 16
NW = NC * NS

C = 128
CHUNKS = 80
EPT = C * CHUNKS
E_PAD = EPT * NW
TRASH = N
N_ACC = 10240
RPT = N_ACC // NS
BPT = RPT // C
NBUF = 2

_mesh = plsc.VectorSubcoreMesh(
    core_axis_name="c", subcore_axis_name="s", num_cores=NC, num_subcores=NS)


def _ids():
    cid = lax.axis_index("c")
    sid = lax.axis_index("s")
    return cid, sid, sid * NC + cid


def _zero_acc(zeros_hbm, stage_v, acc, sid):
    pltpu.sync_copy(zeros_hbm.at[pl.ds(0, C)], stage_v)
    for b in range(BPT):
        pltpu.sync_copy(stage_v, acc.at[pl.ds(sid * RPT + b * C, C)])


def _writeback(acc, stage_v, out_hbm, cid, sid):
    for b in range(BPT):
        r0 = sid * RPT + b * C
        pltpu.sync_copy(acc.at[pl.ds(r0, C)], stage_v)
        pltpu.sync_copy(stage_v, out_hbm.at[pl.ds(cid * N_ACC + r0, C)])


def _preload_cols(colp_hbm, col_b, wid):
    pltpu.sync_copy(colp_hbm.at[pl.ds(wid * CHUNKS, CHUNKS)], col_b)


def _mask_cols(col_b, rbuf, i):
    for j in range(C // 16):
        sl = pl.ds(j * 16, 16)
        r = rbuf[sl]
        c = col_b[i, sl]
        col_b[i, sl] = jnp.where(r == c, TRASH, c)


K0 = 120
K1 = 40
KMAX = max(K0, K1)


def _agg_pipeline(h_hbm, rowp_hbm, acc, col_b, rbufs, rows,
                  isems, gsems, ssems, chunk0, K):
    rounds = K // NBUF

    for b in range(NBUF):
        pltpu.sync_copy(rowp_hbm.at[chunk0 + b], rbufs[b])
        _mask_cols(col_b, rbufs[b], b)
        pltpu.async_copy(h_hbm.at[rbufs[b]], rows[b], gsems[b])

    def steady(o, _):
        for b in range(NBUF):
            ic = o * NBUF + b
            pltpu.make_async_copy(h_hbm.at[rbufs[b]], rows[b],
                                  gsems[b]).wait()
            pltpu.async_copy(rows[b], acc.at[col_b.at[ic]], ssems[b],
                             add=True)
            pltpu.async_copy(rowp_hbm.at[chunk0 + ic + NBUF],
                             rbufs[b], isems[b])
        for b in range(NBUF):
            ic = o * NBUF + b
            pltpu.make_async_copy(rowp_hbm.at[0], rbufs[b],
                                  isems[b]).wait()
            _mask_cols(col_b, rbufs[b], ic + NBUF)
            pltpu.make_async_copy(rows[b], acc.at[col_b.at[ic]],
                                  ssems[b]).wait()
            pltpu.async_copy(h_hbm.at[rbufs[b]], rows[b], gsems[b])
        return ()

    lax.fori_loop(0, rounds - 1, steady, (), unroll=False)

    for b in range(NBUF):
        ic = (rounds - 1) * NBUF + b
        pltpu.make_async_copy(h_hbm.at[rbufs[b]], rows[b], gsems[b]).wait()
        pltpu.async_copy(rows[b], acc.at[col_b.at[ic]], ssems[b], add=True)
    for b in range(NBUF):
        ic = (rounds - 1) * NBUF + b
        pltpu.make_async_copy(rows[b], acc.at[col_b.at[ic]], ssems[b]).wait()


def _sc_agg_body(h_hbm, rowp_hbm, colp_hbm, zeros_hbm, parts_out,
                 acc, col_b, rbufs, rows, isems, gsems, ssems):
    cid, sid, wid = _ids()
    _zero_acc(zeros_hbm, rows[0], acc, sid)
    plsc.subcore_barrier()

    @pl.when(cid == 0)
    def _():
        chunk0 = sid * K0
        pltpu.sync_copy(colp_hbm.at[pl.ds(chunk0, K0)],
                        col_b.at[pl.ds(0, K0)])
        _agg_pipeline(h_hbm, rowp_hbm, acc, col_b, rbufs, rows,
                      isems, gsems, ssems, chunk0, K0)

    @pl.when(cid == 1)
    def _():
        chunk0 = NS * K0 + sid * K1
        pltpu.sync_copy(colp_hbm.at[pl.ds(chunk0, K1)],
                        col_b.at[pl.ds(0, K1)])
        _agg_pipeline(h_hbm, rowp_hbm, acc, col_b, rbufs, rows,
                      isems, gsems, ssems, chunk0, K1)

    plsc.subcore_barrier()
    _writeback(acc, rows[0], parts_out, cid, sid)


def _sc_deg_body(rowp_hbm, colp_hbm, zeros_hbm, ones_hbm, deg_out,
                 acc, row_b, col_b, ones_v, ssems):
    cid, sid, wid = _ids()
    _zero_acc(zeros_hbm, ones_v, acc, sid)
    pltpu.sync_copy(ones_hbm, ones_v)
    pltpu.sync_copy(rowp_hbm.at[pl.ds(wid * CHUNKS, CHUNKS)], row_b)
    _preload_cols(colp_hbm, col_b, wid)

    def mask_all(i, _):
        _mask_cols(col_b, row_b.at[i], i)
        return ()

    lax.fori_loop(0, CHUNKS, mask_all, (), unroll=False)
    plsc.subcore_barrier()

    def steady(o, _):
        for b in range(NBUF):
            i = o * NBUF + b
            pltpu.async_copy(ones_v, acc.at[col_b.at[i]], ssems[b], add=True)
        for b in range(NBUF):
            i = o * NBUF + b
            pltpu.make_async_copy(ones_v, acc.at[col_b.at[i]],
                                  ssems[b]).wait()
        return ()

    lax.fori_loop(0, CHUNKS // NBUF, steady, (), unroll=False)
    plsc.subcore_barrier()
    _writeback(acc, ones_v, deg_out, cid, sid)


_sc_agg = pl.kernel(
    _sc_agg_body,
    out_type=[jax.ShapeDtypeStruct((NC * N_ACC, D), jnp.float32)],
    mesh=_mesh,
    scratch_types=[
        pltpu.VMEM_SHARED((N_ACC, D), jnp.float32),
        pltpu.VMEM((KMAX, C), jnp.int32),
        [pltpu.VMEM((C,), jnp.int32) for _ in range(NBUF)],
        [pltpu.VMEM((C, D), jnp.float32) for _ in range(NBUF)],
        [pltpu.SemaphoreType.DMA for _ in range(NBUF)],
        [pltpu.SemaphoreType.DMA for _ in range(NBUF)],
        [pltpu.SemaphoreType.DMA for _ in range(NBUF)],
    ],
)

_sc_deg = pl.kernel(
    _sc_deg_body,
    out_type=[jax.ShapeDtypeStruct((NC * N_ACC, D), jnp.float32)],
    mesh=_mesh,
    scratch_types=[
        pltpu.VMEM_SHARED((N_ACC, D), jnp.float32),
        pltpu.VMEM((CHUNKS, C), jnp.int32),
        pltpu.VMEM((CHUNKS, C), jnp.int32),
        pltpu.VMEM((C, D), jnp.float32),
        [pltpu.SemaphoreType.DMA for _ in range(NBUF)],
    ],
)


def _tc_body(act, parts_ref, degp_ref, h_ref, wo_ref, b_ref, wr_ref, o_ref):
    deg = degp_ref[0] + degp_ref[1] + 1.0
    dinv = 1.0 / deg
    h = h_ref[...]
    agg = (parts_ref[0] + parts_ref[1] + h) * dinv
    y = (lax.dot_general(agg, wo_ref[...], (((1,), (0,)), ((), ())),
                         preferred_element_type=jnp.float32)
         + b_ref[...]
         + lax.dot_general(h, wr_ref[...], (((1,), (0,)), ((), ())),
                           preferred_element_type=jnp.float32))
    y = jnp.maximum(y, 0.0)
    if act == "relu":
        o_ref[...] = y
    else:
        m = jnp.max(y, axis=-1, keepdims=True)
        z = y - m
        o_ref[...] = z - jnp.log(jnp.sum(jnp.exp(z), axis=-1, keepdims=True))


RB = 1024


def _tc_layer(parts, degp, h, W_out, b_out, W_root, act):
    return pl.pallas_call(
        functools.partial(_tc_body, act),
        grid=(N_ACC // RB,),
        in_specs=[
            pl.BlockSpec((NC, RB, D), lambda i: (0, i, 0)),
            pl.BlockSpec((NC, RB, D), lambda i: (0, i, 0)),
            pl.BlockSpec((RB, D), lambda i: (i, 0)),
            pl.BlockSpec((D, D), lambda i: (0, 0)),
            pl.BlockSpec((1, D), lambda i: (0, 0)),
            pl.BlockSpec((D, D), lambda i: (0, 0)),
        ],
        out_specs=pl.BlockSpec((RB, D), lambda i: (i, 0)),
        out_shape=jax.ShapeDtypeStruct((N_ACC, D), jnp.float32),
    )(parts, degp, h, W_out, b_out.reshape(1, D), W_root)


def kernel(x, edge_index, W_out0, b_out0, W_root0, W_out1, b_out1, W_root1,
           W_out2, b_out2, W_root2):
    row = edge_index[0]
    col = edge_index[1]
    pad = E_PAD - E
    rowp = jnp.concatenate([row, jnp.zeros((pad,), jnp.int32)])
    colp = jnp.concatenate([col, jnp.full((pad,), TRASH, jnp.int32)])
    rowp = rowp.reshape(NW * CHUNKS, C)
    colp = colp.reshape(NW * CHUNKS, C)
    zeros = jnp.zeros((N_ACC, D), jnp.float32)
    ones = jnp.ones((C, D), jnp.float32)
    xp = jnp.concatenate([x, jnp.zeros((N_ACC - N, D), jnp.float32)])

    (degp,) = _sc_deg(rowp, colp, zeros, ones)
    degp = degp.reshape(NC, N_ACC, D)
    (parts0,) = _sc_agg(xp, rowp, colp, zeros)
    parts0 = parts0.reshape(NC, N_ACC, D)
    h1 = _tc_layer(parts0, degp, xp, W_out0, b_out0, W_root0, "relu")
    (parts1,) = _sc_agg(h1, rowp, colp, zeros)
    parts1 = parts1.reshape(NC, N_ACC, D)
    h2 = _tc_layer(parts1, degp, h1, W_out1, b_out1, W_root1, "relu")
    (parts2,) = _sc_agg(h2, rowp, colp, zeros)
    parts2 = parts2.reshape(NC, N_ACC, D)
    out = _tc_layer(parts2, degp, h2, W_out2, b_out2, W_root2, "logsoftmax")
    return out[:N]
